# baseline (speedup 1.0000x reference)
import functools

import jax
import jax.numpy as jnp
from jax import lax
from jax.experimental import pallas as pl
from jax.experimental.pallas import tpu as pltpu
from jax.experimental.pallas import tpu_sc as plsc

_TB = 4096
_P = jax.lax.Precision.DEFAULT
_NW = 32


def _sc_gather(idx3, tpa, tpb, B):
    bpw = B // _NW
    nch = bpw // 128
    mesh = plsc.VectorSubcoreMesh(core_axis_name="c", subcore_axis_name="s")

    @functools.partial(
        pl.kernel, mesh=mesh,
        out_type=jax.ShapeDtypeStruct((2, B, 128), jnp.float32),
        scratch_types=[
            pltpu.VMEM((nch, 128), jnp.int32),
            pltpu.VMEM((nch, 128), jnp.int32),
            pltpu.VMEM((nch, 128), jnp.int32),
            pltpu.VMEM((bpw, 128), jnp.float32),
            pltpu.SemaphoreType.DMA,
        ],
    )
    def k(idx_hbm, ta, tb, out_hbm, ia_v, ib_v, pi_v, rows_v, sem):
        w = lax.axis_index("s") * 2 + lax.axis_index("c")
        base = w * bpw
        for pp, t in enumerate((ta, tb)):
            pltpu.sync_copy(idx_hbm.at[2 * pp, pl.ds(nch * w, nch)], ia_v)
            pltpu.sync_copy(idx_hbm.at[2 * pp + 1, pl.ds(nch * w, nch)], ib_v)
            for r in range(nch):
                for j in range(8):
                    sl = (r, pl.ds(16 * j, 16))
                    pi_v[sl] = ia_v[sl] * 10 + ib_v[sl]
            copies = [
                pltpu.async_copy(t.at[pi_v.at[c]],
                                 rows_v.at[pl.ds(128 * c, 128)], sem)
                for c in range(nch)
            ]
            for cp in copies:
                cp.wait()
            pltpu.sync_copy(rows_v, out_hbm.at[pp, pl.ds(base, bpw)])

    return k(idx3, tpa, tpb)


def _mlp_body(embs_ref, w1, b1_, w2, b2_, w3, b3_, out_ref, w1b, w2b, w3b):
    i = pl.program_id(0)

    @pl.when(i == 0)
    def _cast():
        w1b[...] = w1[...].astype(jnp.bfloat16)
        w2b[...] = w2[...].astype(jnp.bfloat16)
        w3b[...] = w3[...].astype(jnp.bfloat16)

    e4 = embs_ref[...]
    e = jnp.concatenate([e4[0], e4[1]], axis=-1).astype(jnp.bfloat16)
    h = jnp.dot(e, w1b[...], preferred_element_type=jnp.float32,
                precision=_P) + b1_[...][None, :]
    h = jnp.maximum(h.astype(jnp.bfloat16), jnp.bfloat16(0.0))
    h = jnp.dot(h, w2b[...], preferred_element_type=jnp.float32,
                precision=_P) + b2_[...][None, :]
    h = jnp.maximum(h.astype(jnp.bfloat16), jnp.bfloat16(0.0))
    out_ref[...] = jnp.dot(h, w3b[...], preferred_element_type=jnp.float32,
                           precision=_P) + b3_[...][None, :]


def kernel(batch_tuta_feat, mag_table, prec_table, msd_table, lsd_table,
           W1, b1, W2, b2, W3, b3):
    B = batch_tuta_feat.shape[0]
    SUB = mag_table.shape[1]
    HID = W2.shape[0]
    OUTC = W3.shape[1]
    G = B // _TB

    idx3 = batch_tuta_feat.T.reshape(4, B // 128, 128)
    tpa = jnp.concatenate([jnp.repeat(mag_table, 10, axis=0),
                           jnp.tile(prec_table, (10, 1))], axis=1)
    tpb = jnp.concatenate([jnp.repeat(msd_table, 10, axis=0),
                           jnp.tile(lsd_table, (10, 1))], axis=1)
    embs = _sc_gather(idx3, tpa, tpb, B)

    full = lambda shape: pl.BlockSpec(shape, lambda i: (0, 0))
    full1 = lambda n: pl.BlockSpec((n,), lambda i: (0,))
    return pl.pallas_call(
        _mlp_body,
        grid=(G,),
        in_specs=[
            pl.BlockSpec((2, _TB, 128), lambda i: (0, i, 0)),
            full(W1.shape),
            full1(HID),
            full(W2.shape),
            full1(HID),
            full(W3.shape),
            full1(OUTC),
        ],
        out_specs=pl.BlockSpec((_TB, OUTC), lambda i: (i, 0)),
        out_shape=jax.ShapeDtypeStruct((B, OUTC), jnp.float32),
        scratch_shapes=[pltpu.VMEM((4 * SUB, HID), jnp.bfloat16),
                        pltpu.VMEM((HID, HID), jnp.bfloat16),
                        pltpu.VMEM((HID, OUTC), jnp.bfloat16)],
        compiler_params=pltpu.CompilerParams(
            dimension_semantics=("arbitrary",)),
    )(embs, W1, b1, W2, b2, W3, b3)


# speedup vs baseline: 2.1814x; 2.1814x over previous
import jax
import jax.numpy as jnp
from jax.experimental import pallas as pl
from jax.experimental.pallas import tpu as pltpu

_TB = 4096
_P = jax.lax.Precision.DEFAULT


def _mlp_body(idx_ref, mt, pt, st, lt, w1, b1_, w2, b2_, w3, b3_, out_ref,
              tt, w2b, w3b):
    i = pl.program_id(0)

    @pl.when(i == 0)
    def _fold():
        z = jnp.zeros((6, 64), jnp.float32)
        for k, tref in enumerate((mt, pt, st, lt)):
            tab = jnp.concatenate([tref[...], z], axis=0)
            blk = jnp.dot(tab, w1[pl.ds(64 * k, 64), :],
                          preferred_element_type=jnp.float32,
                          precision=jax.lax.Precision.HIGHEST)
            if k == 0:
                row = jax.lax.broadcasted_iota(jnp.int32, (16, 1), 0)
                blk = blk + (row == 15).astype(jnp.float32) * b1_[...][None, :]
            tt[pl.ds(16 * k, 16), :] = blk.astype(jnp.bfloat16)
        w2b[...] = w2[...].astype(jnp.bfloat16)
        w3b[...] = w3[...].astype(jnp.bfloat16)

    half = _TB // 2
    for s in range(2):
        idx = idx_ref[pl.ds(i * _TB + s * half, half), :].astype(jnp.float32)
        gk = jax.lax.broadcasted_iota(jnp.int32, (4, 64), 1) // 16
        e = (gk == jax.lax.broadcasted_iota(jnp.int32, (4, 64), 0))
        idxb = jnp.dot(idx, e.astype(jnp.float32),
                       preferred_element_type=jnp.float32, precision=_P)
        col = jax.lax.broadcasted_iota(jnp.int32, (half, 64), 1)
        o = (jnp.remainder(col, 16).astype(jnp.float32) == idxb) | (col == 15)
        onehot = o.astype(jnp.bfloat16)

        h = jnp.dot(onehot, tt[...], preferred_element_type=jnp.float32,
                    precision=_P)
        h = jnp.maximum(h.astype(jnp.bfloat16), jnp.bfloat16(0.0))
        h = jnp.dot(h, w2b[...], preferred_element_type=jnp.float32,
                    precision=_P) + b2_[...][None, :]
        h = jnp.maximum(h.astype(jnp.bfloat16), jnp.bfloat16(0.0))
        out_ref[pl.ds(s * half, half), :] = jnp.dot(
            h, w3b[...], preferred_element_type=jnp.float32,
            precision=_P) + b3_[...][None, :]


def kernel(batch_tuta_feat, mag_table, prec_table, msd_table, lsd_table,
           W1, b1, W2, b2, W3, b3):
    B = batch_tuta_feat.shape[0]
    HID = W2.shape[0]
    OUTC = W3.shape[1]
    G = B // _TB

    full = lambda shape: pl.BlockSpec(shape, lambda i: (0, 0))
    full1 = lambda n: pl.BlockSpec((n,), lambda i: (0,))
    return pl.pallas_call(
        _mlp_body,
        grid=(G,),
        in_specs=[
            full((B, 4)),
            full(mag_table.shape), full(prec_table.shape),
            full(msd_table.shape), full(lsd_table.shape),
            full(W1.shape),
            full1(HID),
            full(W2.shape),
            full1(HID),
            full(W3.shape),
            full1(OUTC),
        ],
        out_specs=pl.BlockSpec((_TB, OUTC), lambda i: (i, 0)),
        out_shape=jax.ShapeDtypeStruct((B, OUTC), jnp.float32),
        scratch_shapes=[pltpu.VMEM((64, HID), jnp.bfloat16),
                        pltpu.VMEM((HID, HID), jnp.bfloat16),
                        pltpu.VMEM((HID, OUTC), jnp.bfloat16)],
        compiler_params=pltpu.CompilerParams(
            dimension_semantics=("arbitrary",)),
    )(batch_tuta_feat, mag_table, prec_table, msd_table, lsd_table,
      W1, b1, W2, b2, W3, b3)


# speedup vs baseline: 2.1927x; 1.0052x over previous
import jax
import jax.numpy as jnp
from jax.experimental import pallas as pl
from jax.experimental.pallas import tpu as pltpu

_TB = 4096
_P = jax.lax.Precision.DEFAULT


def _mlp_body(idx_ref, mt, pt, st, lt, w1, b1_, w2, b2_, w3, b3_, out_ref,
              tt, w2b, w3b):
    i = pl.program_id(0)

    @pl.when(i == 0)
    def _fold():
        z = jnp.zeros((6, 64), jnp.float32)
        for k, tref in enumerate((mt, pt, st, lt)):
            tab = jnp.concatenate([tref[...], z], axis=0)
            blk = jnp.dot(tab, w1[pl.ds(64 * k, 64), :],
                          preferred_element_type=jnp.float32,
                          precision=jax.lax.Precision.HIGHEST)
            if k == 0:
                row = jax.lax.broadcasted_iota(jnp.int32, (16, 1), 0)
                blk = blk + (row == 15).astype(jnp.float32) * b1_[...][None, :]
            tt[pl.ds(16 * k, 16), :] = blk.astype(jnp.bfloat16)
        w2b[...] = w2[...].astype(jnp.bfloat16)
        w3b[...] = w3[...].astype(jnp.bfloat16)

    idx = idx_ref[pl.ds(i * _TB, _TB), :].astype(jnp.float32)
    gk = jax.lax.broadcasted_iota(jnp.int32, (4, 64), 1) // 16
    e = (gk == jax.lax.broadcasted_iota(jnp.int32, (4, 64), 0))
    idxb = jnp.dot(idx, e.astype(jnp.float32),
                   preferred_element_type=jnp.float32, precision=_P)
    col = jax.lax.broadcasted_iota(jnp.int32, (_TB, 64), 1)
    o = (jnp.remainder(col, 16).astype(jnp.float32) == idxb) | (col == 15)
    onehot = o.astype(jnp.bfloat16)

    h = jnp.dot(onehot, tt[...], preferred_element_type=jnp.float32,
                precision=_P)
    h = jnp.maximum(h.astype(jnp.bfloat16), jnp.bfloat16(0.0))
    h = jnp.dot(h, w2b[...], preferred_element_type=jnp.float32,
                precision=_P) + b2_[...][None, :]
    h = jnp.maximum(h.astype(jnp.bfloat16), jnp.bfloat16(0.0))
    out_ref[...] = jnp.dot(h, w3b[...], preferred_element_type=jnp.float32,
                           precision=_P) + b3_[...][None, :]


def kernel(batch_tuta_feat, mag_table, prec_table, msd_table, lsd_table,
           W1, b1, W2, b2, W3, b3):
    B = batch_tuta_feat.shape[0]
    HID = W2.shape[0]
    OUTC = W3.shape[1]
    G = B // _TB

    full = lambda shape: pl.BlockSpec(shape, lambda i: (0, 0))
    full1 = lambda n: pl.BlockSpec((n,), lambda i: (0,))
    return pl.pallas_call(
        _mlp_body,
        grid=(G,),
        in_specs=[
            full((B, 4)),
            full(mag_table.shape), full(prec_table.shape),
            full(msd_table.shape), full(lsd_table.shape),
            full(W1.shape),
            full1(HID),
            full(W2.shape),
            full1(HID),
            full(W3.shape),
            full1(OUTC),
        ],
        out_specs=pl.BlockSpec((_TB, OUTC), lambda i: (i, 0)),
        out_shape=jax.ShapeDtypeStruct((B, OUTC), jnp.float32),
        scratch_shapes=[pltpu.VMEM((64, HID), jnp.bfloat16),
                        pltpu.VMEM((HID, HID), jnp.bfloat16),
                        pltpu.VMEM((HID, OUTC), jnp.bfloat16)],
        compiler_params=pltpu.CompilerParams(
            dimension_semantics=("arbitrary",)),
    )(batch_tuta_feat, mag_table, prec_table, msd_table, lsd_table,
      W1, b1, W2, b2, W3, b3)
